# baseline (speedup 1.0000x reference)
import functools

import jax
import jax.numpy as jnp
from jax import lax
from jax.experimental import pallas as pl
from jax.experimental.pallas import tpu as pltpu
from jax.experimental.pallas import tpu_sc as plsc

_NC = 2
_NS = 16
_LANES = 16


def _make_sc_kernel(bs, flat):
    nw = _NC * _NS
    b_per_w = bs // nw
    rep = 4
    n_dma = b_per_w // rep
    mesh = plsc.VectorSubcoreMesh(core_axis_name="c", subcore_axis_name="s")

    @functools.partial(
        pl.kernel,
        mesh=mesh,
        out_type=jax.ShapeDtypeStruct((bs, flat), jnp.float32),
        scratch_types=[
            pltpu.VMEM((rep, flat), jnp.float32),
            pltpu.SemaphoreType.DMA,
        ],
    )
    def sc_k(qpw_hbm, q_hbm, out_hbm, rep_v, sem):
        wid = lax.axis_index("s") * _NC + lax.axis_index("c")
        pltpu.sync_copy(qpw_hbm, rep_v.at[0])
        pltpu.sync_copy(q_hbm, rep_v.at[1])

        def add_body(i, carry):
            sl = pl.ds(i * _LANES, _LANES)
            v = rep_v[0, sl] + rep_v[1, sl]
            rep_v[0, sl] = v
            rep_v[1, sl] = v
            rep_v[2, sl] = v
            rep_v[3, sl] = v
            return carry

        lax.fori_loop(0, flat // _LANES, add_body, 0)
        base = wid * b_per_w
        copies = [
            pltpu.make_async_copy(
                rep_v, out_hbm.at[pl.ds(base + j * rep, rep)], sem
            )
            for j in range(n_dma)
        ]
        for c in copies:
            c.start()
        for c in copies:
            c.wait()

    return sc_k


def kernel(x, query_pos_weight, queries):
    bs = x.shape[0]
    n_query, embed_dim = query_pos_weight.shape
    flat = n_query * embed_dim
    qpw = query_pos_weight.reshape(flat)
    q = queries.reshape(flat)
    out = _make_sc_kernel(bs, flat)(qpw, q)
    return out.reshape(bs, n_query, embed_dim)


# speedup vs baseline: 6.7511x; 6.7511x over previous
import jax
import jax.numpy as jnp
from jax.experimental import pallas as pl

_B_BLK = 64


def _bcast_add_kernel(qpw_ref, q_ref, out_ref):
    s = q_ref[0] + qpw_ref[...]
    out_ref[...] = jnp.broadcast_to(s[:, None, :], out_ref.shape)


def kernel(x, query_pos_weight, queries):
    bs = x.shape[0]
    n_query, embed_dim = query_pos_weight.shape
    grid = (bs // _B_BLK,)
    out = pl.pallas_call(
        _bcast_add_kernel,
        grid=grid,
        in_specs=[
            pl.BlockSpec((n_query, embed_dim), lambda i: (0, 0)),
            pl.BlockSpec((1, n_query, embed_dim), lambda i: (0, 0, 0)),
        ],
        out_specs=pl.BlockSpec((n_query, _B_BLK, embed_dim), lambda i: (0, i, 0)),
        out_shape=jax.ShapeDtypeStruct((n_query, bs, embed_dim), queries.dtype),
    )(query_pos_weight, queries)
    return jnp.swapaxes(out, 0, 1)
